# baseline (speedup 1.0000x reference)
import jax
import jax.numpy as jnp
from jax.experimental import pallas as pl


def _fused_norm_matmul_kernel(x_ref, t_ref, o_ref):
    x = x_ref[...]
    count = jnp.sum(x, axis=1, keepdims=True)
    count = jnp.where(count == 0.0, 1.0, count)
    acc = jnp.dot(
        x.astype(jnp.bfloat16),
        t_ref[...].astype(jnp.bfloat16),
        preferred_element_type=jnp.float32,
    )
    o_ref[...] = acc / count


def kernel(concept_seq, table, domain):
    M, K = concept_seq.shape
    Kt, N = table.shape
    BM = 1024
    grid = (M // BM,)
    out = pl.pallas_call(
        _fused_norm_matmul_kernel,
        grid=grid,
        in_specs=[
            pl.BlockSpec((BM, K), lambda i: (i, 0)),
            pl.BlockSpec((Kt, N), lambda i: (0, 0)),
        ],
        out_specs=pl.BlockSpec((BM, N), lambda i: (i, 0)),
        out_shape=jax.ShapeDtypeStruct((M, N), jnp.float32),
    )(concept_seq, table)
    return out


# speedup vs baseline: 1.0656x; 1.0656x over previous
import jax
import jax.numpy as jnp
from jax.experimental import pallas as pl


def _fused_norm_matmul_kernel(x_ref, t_ref, o_ref):
    x = x_ref[...]
    count = jnp.sum(x, axis=1, keepdims=True)
    count = jnp.where(count == 0.0, 1.0, count)
    acc = jnp.dot(
        x.astype(jnp.bfloat16),
        t_ref[...].astype(jnp.bfloat16),
        preferred_element_type=jnp.float32,
    )
    o_ref[...] = acc / count


def kernel(concept_seq, table, domain):
    M, K = concept_seq.shape
    Kt, N = table.shape
    BM = 2048
    grid = (M // BM,)
    out = pl.pallas_call(
        _fused_norm_matmul_kernel,
        grid=grid,
        in_specs=[
            pl.BlockSpec((BM, K), lambda i: (i, 0)),
            pl.BlockSpec((Kt, N), lambda i: (0, 0)),
        ],
        out_specs=pl.BlockSpec((BM, N), lambda i: (i, 0)),
        out_shape=jax.ShapeDtypeStruct((M, N), jnp.float32),
    )(concept_seq, table)
    return out
